# Initial kernel scaffold; baseline (speedup 1.0000x reference)
import functools

import jax
import jax.numpy as jnp
from jax import lax
from jax.experimental import pallas as pl
from jax.experimental.pallas import tpu as pltpu
from jax.experimental.pallas import tpu_sc as plsc

NUM_CORES = 2
NUM_SUBCORES = 16
NW = NUM_CORES * NUM_SUBCORES
CHUNK = 128


def _sc_segment_sums(n_acc, n_chunks, d_feat, d_edge, x, src3, dst3, ef4,
                     ones_rows, zer_x, zer_e):
    stripe = n_acc // NUM_SUBCORES
    mesh = plsc.VectorSubcoreMesh(core_axis_name="c", subcore_axis_name="s")

    @functools.partial(
        pl.kernel,
        out_type=[
            jax.ShapeDtypeStruct((NUM_CORES, n_acc, d_feat), jnp.float32),
            jax.ShapeDtypeStruct((NUM_CORES, n_acc, 16), jnp.float32),
            jax.ShapeDtypeStruct((NUM_CORES, n_acc, 16), jnp.float32),
        ],
        mesh=mesh,
        scratch_types=[
            pltpu.VMEM((n_chunks, CHUNK), jnp.int32),
            pltpu.VMEM((n_chunks, CHUNK), jnp.int32),
            pltpu.VMEM((CHUNK, d_feat), jnp.float32),
            pltpu.VMEM((CHUNK, d_edge), jnp.float32),
            pltpu.VMEM((CHUNK, 16), jnp.float32),
            pltpu.VMEM_SHARED((n_acc, d_feat), jnp.float32),
            pltpu.VMEM_SHARED((n_acc, 16), jnp.float32),
            pltpu.VMEM_SHARED((n_acc, 16), jnp.float32),
            pltpu.SemaphoreType.DMA,
        ],
    )
    def sc_kernel(x_hbm, src_hbm, dst_hbm, ef_hbm, ones_hbm, zx_hbm, ze_hbm,
                  outx_hbm, oute_hbm, outc_hbm,
                  src_v, dst_v, xbuf, efbuf, onesbuf, acc_x, acc_e, acc_c,
                  sem):
        c = lax.axis_index("c")
        s = lax.axis_index("s")
        w = c * NUM_SUBCORES + s
        base = s * stripe

        pltpu.sync_copy(zx_hbm, acc_x.at[pl.ds(base, stripe)])
        pltpu.sync_copy(ze_hbm, acc_e.at[pl.ds(base, stripe)])
        pltpu.sync_copy(ze_hbm, acc_c.at[pl.ds(base, stripe)])
        pltpu.sync_copy(src_hbm.at[w], src_v)
        pltpu.sync_copy(dst_hbm.at[w], dst_v)
        pltpu.sync_copy(ones_hbm, onesbuf)
        plsc.subcore_barrier()

        def body(j, carry):
            pltpu.async_copy(x_hbm.at[src_v.at[j]], xbuf, sem).wait()
            pltpu.sync_copy(ef_hbm.at[w, j], efbuf)
            pltpu.sync_copy(xbuf, acc_x.at[dst_v.at[j]], add=True)
            pltpu.sync_copy(efbuf, acc_e.at[dst_v.at[j]], add=True)
            pltpu.sync_copy(onesbuf, acc_c.at[dst_v.at[j]], add=True)
            return carry

        lax.fori_loop(0, n_chunks, body, 0)
        plsc.subcore_barrier()

        pltpu.sync_copy(acc_x.at[pl.ds(base, stripe)],
                        outx_hbm.at[c, pl.ds(base, stripe)])
        pltpu.sync_copy(acc_e.at[pl.ds(base, stripe)],
                        oute_hbm.at[c, pl.ds(base, stripe)])
        pltpu.sync_copy(acc_c.at[pl.ds(base, stripe)],
                        outc_hbm.at[c, pl.ds(base, stripe)])

    return sc_kernel(x, src3, dst3, ef4, ones_rows, zer_x, zer_e)


def _tc_body(d_feat, px_ref, pe_ref, pc_ref, wt_ref, b_ref, out_ref):
    sx = px_ref[0] + px_ref[1]
    se = pe_ref[0] + pe_ref[1]
    cnt = (pc_ref[0] + pc_ref[1])[:, 0:1]
    acc = jnp.dot(sx, wt_ref[:d_feat],
                  preferred_element_type=jnp.float32,
                  precision=lax.Precision.HIGHEST)
    acc = acc + jnp.dot(se, wt_ref[d_feat:],
                        preferred_element_type=jnp.float32,
                        precision=lax.Precision.HIGHEST)
    acc = acc + cnt * b_ref
    out_ref[...] = acc / jnp.maximum(cnt, 1.0)


def kernel(x, edge_index, edge_features, W, b):
    n_nodes, d_feat = x.shape
    n_edges = edge_index.shape[1]
    d_edge = edge_features.shape[1]
    out_dim = W.shape[0]

    ep = -(-n_edges // (NW * CHUNK)) * (NW * CHUNK)
    per_tile = ep // NW
    n_chunks = per_tile // CHUNK
    pad = ep - n_edges
    n_acc = -(-(n_nodes + 1) // 1280) * 1280
    stripe = n_acc // NUM_SUBCORES

    src = edge_index[0].astype(jnp.int32)
    dst = edge_index[1].astype(jnp.int32)
    src3 = jnp.concatenate(
        [src, jnp.zeros((pad,), jnp.int32)]).reshape(NW, n_chunks, CHUNK)
    dst3 = jnp.concatenate(
        [dst, jnp.full((pad,), n_nodes, jnp.int32)]).reshape(NW, n_chunks, CHUNK)
    ef4 = jnp.concatenate(
        [edge_features.astype(jnp.float32),
         jnp.zeros((pad, d_edge), jnp.float32)]).reshape(NW, n_chunks, CHUNK, d_edge)
    ones_rows = jnp.zeros((CHUNK, 16), jnp.float32).at[:, 0].set(1.0)
    zer_x = jnp.zeros((stripe, d_feat), jnp.float32)
    zer_e = jnp.zeros((stripe, 16), jnp.float32)

    px, pe, pc = _sc_segment_sums(n_acc, n_chunks, d_feat, d_edge,
                                  x.astype(jnp.float32), src3, dst3, ef4,
                                  ones_rows, zer_x, zer_e)

    wt = W.T.astype(jnp.float32)
    b2 = b.astype(jnp.float32).reshape(1, out_dim)

    blk = 1024
    grid = n_acc // blk
    out_full = pl.pallas_call(
        functools.partial(_tc_body, d_feat),
        grid=(grid,),
        in_specs=[
            pl.BlockSpec((NUM_CORES, blk, d_feat), lambda i: (0, i, 0)),
            pl.BlockSpec((NUM_CORES, blk, 16), lambda i: (0, i, 0)),
            pl.BlockSpec((NUM_CORES, blk, 16), lambda i: (0, i, 0)),
            pl.BlockSpec((d_feat + d_edge, out_dim), lambda i: (0, 0)),
            pl.BlockSpec((1, out_dim), lambda i: (0, 0)),
        ],
        out_specs=pl.BlockSpec((blk, out_dim), lambda i: (i, 0)),
        out_shape=jax.ShapeDtypeStruct((n_acc, out_dim), jnp.float32),
    )(px, pe, pc, wt, b2)

    return out_full[:n_nodes]


# speedup vs baseline: 2.8509x; 2.8509x over previous
import functools

import jax
import jax.numpy as jnp
from jax import lax
from jax.experimental import pallas as pl
from jax.experimental.pallas import tpu as pltpu
from jax.experimental.pallas import tpu_sc as plsc

NUM_CORES = 2
NUM_SUBCORES = 16
CHUNK = 128
BLK = 4


def _sc_segment_sums(n_acc, n_blocks, d_half, d_edge, xcat, src4, dst3, ef4,
                     ones_rows, zer_x, zer_e):
    stripe = n_acc // NUM_SUBCORES
    mesh = plsc.VectorSubcoreMesh(core_axis_name="c", subcore_axis_name="s")

    @functools.partial(
        pl.kernel,
        out_type=[
            jax.ShapeDtypeStruct((NUM_CORES, n_acc, d_half), jnp.float32),
            jax.ShapeDtypeStruct((NUM_CORES, n_acc, 16), jnp.float32),
        ],
        mesh=mesh,
        compiler_params=pltpu.CompilerParams(use_tc_tiling_on_sc=False),
        scratch_types=[
            pltpu.VMEM((BLK, CHUNK), jnp.int32),
            pltpu.VMEM((BLK, CHUNK), jnp.int32),
            pltpu.VMEM((CHUNK, d_half), jnp.float32),
            pltpu.VMEM((CHUNK, d_edge), jnp.float32),
            pltpu.VMEM((CHUNK, 16), jnp.float32),
            pltpu.VMEM_SHARED((n_acc, d_half), jnp.float32),
            pltpu.VMEM_SHARED((n_acc, 16), jnp.float32),
            pltpu.SemaphoreType.DMA,
        ],
    )
    def sc_kernel(x_hbm, src_hbm, dst_hbm, ef_hbm, ones_hbm, zx_hbm, ze_hbm,
                  outx_hbm, outa_hbm,
                  src_v, dst_v, xbuf, efbuf, onesbuf, acc_x, acc_a, sem):
        c = lax.axis_index("c")
        s = lax.axis_index("s")
        base = s * stripe

        pltpu.sync_copy(zx_hbm, acc_x.at[pl.ds(base, stripe)])
        pltpu.sync_copy(ze_hbm, acc_a.at[pl.ds(base, stripe)])
        pltpu.sync_copy(ones_hbm, onesbuf)
        plsc.subcore_barrier()

        def body(blk, carry):
            pltpu.sync_copy(src_hbm.at[c, s, pl.ds(blk * BLK, BLK)], src_v)
            pltpu.sync_copy(dst_hbm.at[s, pl.ds(blk * BLK, BLK)], dst_v)
            for j in range(BLK):
                pltpu.async_copy(x_hbm.at[src_v.at[j]], xbuf, sem).wait()
                pltpu.sync_copy(xbuf, acc_x.at[dst_v.at[j]], add=True)

                @pl.when(c == 0)
                def _():
                    pltpu.sync_copy(ef_hbm.at[s, blk * BLK + j], efbuf)
                    pltpu.sync_copy(efbuf, acc_a.at[dst_v.at[j]], add=True)

                @pl.when(c == 1)
                def _():
                    pltpu.sync_copy(onesbuf, acc_a.at[dst_v.at[j]], add=True)

            return carry

        lax.fori_loop(0, n_blocks, body, 0)
        plsc.subcore_barrier()

        pltpu.sync_copy(acc_x.at[pl.ds(base, stripe)],
                        outx_hbm.at[c, pl.ds(base, stripe)])
        pltpu.sync_copy(acc_a.at[pl.ds(base, stripe)],
                        outa_hbm.at[c, pl.ds(base, stripe)])

    return sc_kernel(xcat, src4, dst3, ef4, ones_rows, zer_x, zer_e)


def _tc_body(d_half, px_ref, pa_ref, wt_ref, b_ref, out_ref):
    se = pa_ref[0]
    cnt = pa_ref[1][:, 0:1]
    acc = jnp.dot(px_ref[0], wt_ref[:d_half],
                  preferred_element_type=jnp.float32,
                  precision=lax.Precision.HIGHEST)
    acc = acc + jnp.dot(px_ref[1], wt_ref[d_half:2 * d_half],
                        preferred_element_type=jnp.float32,
                        precision=lax.Precision.HIGHEST)
    acc = acc + jnp.dot(se, wt_ref[2 * d_half:],
                        preferred_element_type=jnp.float32,
                        precision=lax.Precision.HIGHEST)
    acc = acc + cnt * b_ref[...]
    out_ref[...] = acc / jnp.maximum(cnt, 1.0)


def kernel(x, edge_index, edge_features, W, b):
    n_nodes, d_feat = x.shape
    n_edges = edge_index.shape[1]
    d_edge = edge_features.shape[1]
    out_dim = W.shape[0]
    d_half = d_feat // 2

    tile_quant = CHUNK * BLK
    per_tile = -(-n_edges // (NUM_SUBCORES * tile_quant)) * tile_quant
    ep = per_tile * NUM_SUBCORES
    n_blocks = per_tile // tile_quant
    n_chunks = per_tile // CHUNK
    pad = ep - n_edges
    n_acc = -(-(n_nodes + 1) // 1280) * 1280
    stripe = n_acc // NUM_SUBCORES

    src = edge_index[0].astype(jnp.int32)
    dst = edge_index[1].astype(jnp.int32)
    src_p = jnp.concatenate([src, jnp.zeros((pad,), jnp.int32)])
    src4 = jnp.stack([src_p, src_p + n_nodes]).reshape(
        NUM_CORES, NUM_SUBCORES, n_chunks, CHUNK)
    dst3 = jnp.concatenate(
        [dst, jnp.full((pad,), n_nodes, jnp.int32)]).reshape(
        NUM_SUBCORES, n_chunks, CHUNK)
    ef4 = jnp.concatenate(
        [edge_features.astype(jnp.float32),
         jnp.zeros((pad, d_edge), jnp.float32)]).reshape(
        NUM_SUBCORES, n_chunks, CHUNK, d_edge)
    xcat = jnp.concatenate([x[:, :d_half], x[:, d_half:]], axis=0)
    ones_rows = jnp.zeros((CHUNK, 16), jnp.float32).at[:, 0].set(1.0)
    zer_x = jnp.zeros((stripe, d_half), jnp.float32)
    zer_e = jnp.zeros((stripe, 16), jnp.float32)

    px, pa = _sc_segment_sums(n_acc, n_blocks, d_half, d_edge,
                              xcat.astype(jnp.float32), src4, dst3, ef4,
                              ones_rows, zer_x, zer_e)

    wt = W.T.astype(jnp.float32)
    b2 = b.astype(jnp.float32).reshape(1, out_dim)

    blk = 1024
    grid = n_acc // blk
    out_full = pl.pallas_call(
        functools.partial(_tc_body, d_half),
        grid=(grid,),
        in_specs=[
            pl.BlockSpec((NUM_CORES, blk, d_half), lambda i: (0, i, 0)),
            pl.BlockSpec((NUM_CORES, blk, 16), lambda i: (0, i, 0)),
            pl.BlockSpec((d_feat + d_edge, out_dim), lambda i: (0, 0)),
            pl.BlockSpec((1, out_dim), lambda i: (0, 0)),
        ],
        out_specs=pl.BlockSpec((blk, out_dim), lambda i: (i, 0)),
        out_shape=jax.ShapeDtypeStruct((n_acc, out_dim), jnp.float32),
    )(px, pa, wt, b2)

    return out_full[:n_nodes]


# speedup vs baseline: 4.4375x; 1.5565x over previous
import functools

import jax
import jax.numpy as jnp
from jax import lax
from jax.experimental import pallas as pl
from jax.experimental.pallas import tpu as pltpu
from jax.experimental.pallas import tpu_sc as plsc

NUM_CORES = 2
NUM_SUBCORES = 16
CHUNK = 128


def _sc_segment_sums(n_acc, n_chunks, d_half, d_edge, xcat, src4, dst3, ef4,
                     ones_rows, zer_x, zer_e):
    stripe = n_acc // NUM_SUBCORES
    npairs = n_chunks // 2
    mesh = plsc.VectorSubcoreMesh(core_axis_name="c", subcore_axis_name="s")

    @functools.partial(
        pl.kernel,
        out_type=[
            jax.ShapeDtypeStruct((NUM_CORES, n_acc, d_half), jnp.float32),
            jax.ShapeDtypeStruct((NUM_CORES, n_acc, 16), jnp.float32),
        ],
        mesh=mesh,
        compiler_params=pltpu.CompilerParams(use_tc_tiling_on_sc=False),
        scratch_types=[
            pltpu.VMEM((n_chunks, CHUNK), jnp.int32),
            pltpu.VMEM((n_chunks, CHUNK), jnp.int32),
            pltpu.VMEM((CHUNK, d_half), jnp.float32),
            pltpu.VMEM((CHUNK, d_half), jnp.float32),
            pltpu.VMEM((CHUNK, d_edge), jnp.float32),
            pltpu.VMEM((CHUNK, d_edge), jnp.float32),
            pltpu.VMEM((CHUNK, 16), jnp.float32),
            pltpu.VMEM_SHARED((n_acc, d_half), jnp.float32),
            pltpu.VMEM_SHARED((n_acc, 16), jnp.float32),
            pltpu.SemaphoreType.DMA,
            pltpu.SemaphoreType.DMA,
            pltpu.SemaphoreType.DMA,
            pltpu.SemaphoreType.DMA,
            pltpu.SemaphoreType.DMA,
            pltpu.SemaphoreType.DMA,
            pltpu.SemaphoreType.DMA,
            pltpu.SemaphoreType.DMA,
        ],
    )
    def sc_kernel(x_hbm, src_hbm, dst_hbm, ef_hbm, ones_hbm, zx_hbm, ze_hbm,
                  outx_hbm, outa_hbm,
                  src_v, dst_v, xb0, xb1, eb0, eb1, onesbuf, acc_x, acc_a,
                  gx0, gx1, sx0, sx1, el0, el1, ea0, ea1):
        c = lax.axis_index("c")
        s = lax.axis_index("s")
        base = s * stripe

        pltpu.sync_copy(zx_hbm, acc_x.at[pl.ds(base, stripe)])
        pltpu.sync_copy(ze_hbm, acc_a.at[pl.ds(base, stripe)])
        pltpu.sync_copy(ones_hbm, onesbuf)
        pltpu.sync_copy(src_hbm.at[c, s], src_v)
        pltpu.sync_copy(dst_hbm.at[s], dst_v)
        plsc.subcore_barrier()

        def gather_x(j, buf, sem):
            pltpu.async_copy(x_hbm.at[src_v.at[j]], buf, sem)

        def wait_gather_x(j, buf, sem):
            pltpu.make_async_copy(x_hbm.at[src_v.at[j]], buf, sem).wait()

        def scat_x(j, buf, sem):
            pltpu.async_copy(buf, acc_x.at[dst_v.at[j]], sem, add=True)

        def wait_scat_x(j, buf, sem):
            pltpu.make_async_copy(buf, acc_x.at[dst_v.at[j]], sem).wait()

        def load_ef(j, buf, sem):
            pltpu.async_copy(ef_hbm.at[s, j], buf, sem)

        def wait_load_ef(j, buf, sem):
            pltpu.make_async_copy(ef_hbm.at[s, j], buf, sem).wait()

        def scat_aux(j, buf, sem):
            pltpu.async_copy(buf, acc_a.at[dst_v.at[j]], sem, add=True)

        def wait_scat_aux(j, buf, sem):
            pltpu.make_async_copy(buf, acc_a.at[dst_v.at[j]], sem).wait()

        gather_x(0, xb0, gx0)

        @pl.when(c == 0)
        def _():
            load_ef(0, eb0, el0)

        def body(p, carry):
            a = 2 * p
            bch = a + 1

            wait_gather_x(a, xb0, gx0)
            scat_x(a, xb0, sx0)

            @pl.when(c == 0)
            def _():
                wait_load_ef(a, eb0, el0)
                scat_aux(a, eb0, ea0)

            @pl.when(c != 0)
            def _():
                @pl.when(p > 0)
                def _():
                    wait_scat_aux(a, onesbuf, ea0)

                scat_aux(a, onesbuf, ea0)

            @pl.when(p > 0)
            def _():
                wait_scat_x(bch, xb1, sx1)

            gather_x(bch, xb1, gx1)

            @pl.when(c == 0)
            def _():
                @pl.when(p > 0)
                def _():
                    wait_scat_aux(bch, eb1, ea1)

                load_ef(bch, eb1, el1)

            wait_gather_x(bch, xb1, gx1)
            scat_x(bch, xb1, sx1)

            @pl.when(c == 0)
            def _():
                wait_load_ef(bch, eb1, el1)
                scat_aux(bch, eb1, ea1)

            @pl.when(c != 0)
            def _():
                @pl.when(p > 0)
                def _():
                    wait_scat_aux(bch, onesbuf, ea1)

                scat_aux(bch, onesbuf, ea1)

            @pl.when(p < npairs - 1)
            def _():
                wait_scat_x(a, xb0, sx0)
                gather_x(a + 2, xb0, gx0)

                @pl.when(c == 0)
                def _():
                    wait_scat_aux(a, eb0, ea0)
                    load_ef(a + 2, eb0, el0)

            return carry

        lax.fori_loop(0, npairs, body, 0)

        wait_scat_x(n_chunks - 2, xb0, sx0)
        wait_scat_x(n_chunks - 1, xb1, sx1)

        @pl.when(c == 0)
        def _():
            wait_scat_aux(n_chunks - 2, eb0, ea0)
            wait_scat_aux(n_chunks - 1, eb1, ea1)

        @pl.when(c != 0)
        def _():
            wait_scat_aux(n_chunks - 2, onesbuf, ea0)
            wait_scat_aux(n_chunks - 1, onesbuf, ea1)

        plsc.subcore_barrier()

        pltpu.sync_copy(acc_x.at[pl.ds(base, stripe)],
                        outx_hbm.at[c, pl.ds(base, stripe)])
        pltpu.sync_copy(acc_a.at[pl.ds(base, stripe)],
                        outa_hbm.at[c, pl.ds(base, stripe)])

    return sc_kernel(xcat, src4, dst3, ef4, ones_rows, zer_x, zer_e)


def _tc_body(d_half, px_ref, pa_ref, wt_ref, b_ref, out_ref):
    se = pa_ref[0]
    cnt = pa_ref[1][:, 0:1]
    acc = jnp.dot(px_ref[0], wt_ref[:d_half],
                  preferred_element_type=jnp.float32,
                  precision=lax.Precision.HIGHEST)
    acc = acc + jnp.dot(px_ref[1], wt_ref[d_half:2 * d_half],
                        preferred_element_type=jnp.float32,
                        precision=lax.Precision.HIGHEST)
    acc = acc + jnp.dot(se, wt_ref[2 * d_half:],
                        preferred_element_type=jnp.float32,
                        precision=lax.Precision.HIGHEST)
    acc = acc + cnt * b_ref[...]
    out_ref[...] = acc / jnp.maximum(cnt, 1.0)


def kernel(x, edge_index, edge_features, W, b):
    n_nodes, d_feat = x.shape
    n_edges = edge_index.shape[1]
    d_edge = edge_features.shape[1]
    out_dim = W.shape[0]
    d_half = d_feat // 2

    tile_quant = 2 * CHUNK
    per_tile = -(-n_edges // (NUM_SUBCORES * tile_quant)) * tile_quant
    ep = per_tile * NUM_SUBCORES
    n_chunks = per_tile // CHUNK
    pad = ep - n_edges
    n_acc = -(-(n_nodes + 1) // 1280) * 1280
    stripe = n_acc // NUM_SUBCORES

    src = edge_index[0].astype(jnp.int32)
    dst = edge_index[1].astype(jnp.int32)
    src_p = jnp.concatenate([src, jnp.zeros((pad,), jnp.int32)])
    src4 = jnp.stack([src_p, src_p + n_nodes]).reshape(
        NUM_CORES, NUM_SUBCORES, n_chunks, CHUNK)
    dst3 = jnp.concatenate(
        [dst, jnp.full((pad,), n_nodes, jnp.int32)]).reshape(
        NUM_SUBCORES, n_chunks, CHUNK)
    ef4 = jnp.concatenate(
        [edge_features.astype(jnp.float32),
         jnp.zeros((pad, d_edge), jnp.float32)]).reshape(
        NUM_SUBCORES, n_chunks, CHUNK, d_edge)
    xcat = jnp.concatenate([x[:, :d_half], x[:, d_half:]], axis=0)
    ones_rows = jnp.zeros((CHUNK, 16), jnp.float32).at[:, 0].set(1.0)
    zer_x = jnp.zeros((stripe, d_half), jnp.float32)
    zer_e = jnp.zeros((stripe, 16), jnp.float32)

    px, pa = _sc_segment_sums(n_acc, n_chunks, d_half, d_edge,
                              xcat.astype(jnp.float32), src4, dst3, ef4,
                              ones_rows, zer_x, zer_e)

    wt = W.T.astype(jnp.float32)
    b2 = b.astype(jnp.float32).reshape(1, out_dim)

    blk = 1024
    grid = n_acc // blk
    out_full = pl.pallas_call(
        functools.partial(_tc_body, d_half),
        grid=(grid,),
        in_specs=[
            pl.BlockSpec((NUM_CORES, blk, d_half), lambda i: (0, i, 0)),
            pl.BlockSpec((NUM_CORES, blk, 16), lambda i: (0, i, 0)),
            pl.BlockSpec((d_feat + d_edge, out_dim), lambda i: (0, 0)),
            pl.BlockSpec((1, out_dim), lambda i: (0, 0)),
        ],
        out_specs=pl.BlockSpec((blk, out_dim), lambda i: (i, 0)),
        out_shape=jax.ShapeDtypeStruct((n_acc, out_dim), jnp.float32),
    )(px, pa, wt, b2)

    return out_full[:n_nodes]


# speedup vs baseline: 5.4279x; 1.2232x over previous
import functools

import jax
import jax.numpy as jnp
from jax import lax
from jax.experimental import pallas as pl
from jax.experimental.pallas import tpu as pltpu
from jax.experimental.pallas import tpu_sc as plsc

NUM_CORES = 2
NUM_SUBCORES = 16
CHUNK = 128


def _sc_segment_sums(n_acc, n_chunks, n_real_chunks, d_half, d_edge, xcat,
                     src3, dst3, ef2, ones_rows, zer_x, zer_e):
    stripe = n_acc // NUM_SUBCORES
    npairs = n_chunks // 2
    mesh = plsc.VectorSubcoreMesh(core_axis_name="c", subcore_axis_name="s")

    @functools.partial(
        pl.kernel,
        out_type=[
            jax.ShapeDtypeStruct((NUM_CORES, n_acc, d_half), jnp.float32),
            jax.ShapeDtypeStruct((NUM_CORES, n_acc, 16), jnp.float32),
        ],
        mesh=mesh,
        compiler_params=pltpu.CompilerParams(use_tc_tiling_on_sc=False),
        scratch_types=[
            pltpu.VMEM((n_chunks, CHUNK), jnp.int32),
            pltpu.VMEM((n_chunks, CHUNK), jnp.int32),
            pltpu.VMEM((CHUNK, d_half), jnp.float32),
            pltpu.VMEM((CHUNK, d_half), jnp.float32),
            pltpu.VMEM((CHUNK, d_edge), jnp.float32),
            pltpu.VMEM((CHUNK, d_edge), jnp.float32),
            pltpu.VMEM((CHUNK, 16), jnp.float32),
            pltpu.VMEM_SHARED((n_acc, d_half), jnp.float32),
            pltpu.VMEM_SHARED((n_acc, 16), jnp.float32),
            pltpu.SemaphoreType.DMA,
            pltpu.SemaphoreType.DMA,
            pltpu.SemaphoreType.DMA,
            pltpu.SemaphoreType.DMA,
            pltpu.SemaphoreType.DMA,
            pltpu.SemaphoreType.DMA,
            pltpu.SemaphoreType.DMA,
            pltpu.SemaphoreType.DMA,
        ],
    )
    def sc_kernel(x_hbm, src_hbm, dst_hbm, ef_hbm, ones_hbm, zx_hbm, ze_hbm,
                  outx_hbm, outa_hbm,
                  src_v, dst_v, xb0, xb1, eb0, eb1, onesbuf, acc_x, acc_a,
                  gx0, gx1, sx0, sx1, el0, el1, ea0, ea1):
        c = lax.axis_index("c")
        s = lax.axis_index("s")
        base = s * stripe

        pltpu.sync_copy(zx_hbm, acc_x.at[pl.ds(base, stripe)])
        pltpu.sync_copy(ze_hbm, acc_a.at[pl.ds(base, stripe)])
        pltpu.sync_copy(ones_hbm, onesbuf)
        pltpu.sync_copy(src_hbm.at[c, pl.ds(s * n_chunks, n_chunks)], src_v)
        pltpu.sync_copy(dst_hbm.at[pl.ds(s * n_chunks, n_chunks)], dst_v)
        plsc.subcore_barrier()

        def ef_rows(j):
            g = jnp.minimum(s * n_chunks + j, n_real_chunks - 1)
            return ef_hbm.at[pl.ds(g * CHUNK, CHUNK)]

        def gather_x(j, buf, sem):
            pltpu.async_copy(x_hbm.at[src_v.at[j]], buf, sem)

        def wait_gather_x(j, buf, sem):
            pltpu.make_async_copy(x_hbm.at[src_v.at[j]], buf, sem).wait()

        def scat_x(j, buf, sem):
            pltpu.async_copy(buf, acc_x.at[dst_v.at[j]], sem, add=True)

        def wait_scat_x(j, buf, sem):
            pltpu.make_async_copy(buf, acc_x.at[dst_v.at[j]], sem).wait()

        def load_ef(j, buf, sem):
            pltpu.async_copy(ef_rows(j), buf, sem)

        def wait_load_ef(j, buf, sem):
            pltpu.make_async_copy(ef_rows(j), buf, sem).wait()

        def scat_aux(j, buf, sem):
            pltpu.async_copy(buf, acc_a.at[dst_v.at[j]], sem, add=True)

        def wait_scat_aux(j, buf, sem):
            pltpu.make_async_copy(buf, acc_a.at[dst_v.at[j]], sem).wait()

        gather_x(0, xb0, gx0)

        @pl.when(c == 0)
        def _():
            load_ef(0, eb0, el0)

        def body(p, carry):
            a = 2 * p
            bch = a + 1

            wait_gather_x(a, xb0, gx0)
            scat_x(a, xb0, sx0)

            @pl.when(c == 0)
            def _():
                wait_load_ef(a, eb0, el0)
                scat_aux(a, eb0, ea0)

            @pl.when(c != 0)
            def _():
                @pl.when(p > 0)
                def _():
                    wait_scat_aux(a, onesbuf, ea0)

                scat_aux(a, onesbuf, ea0)

            @pl.when(p > 0)
            def _():
                wait_scat_x(bch, xb1, sx1)

            gather_x(bch, xb1, gx1)

            @pl.when(c == 0)
            def _():
                @pl.when(p > 0)
                def _():
                    wait_scat_aux(bch, eb1, ea1)

                load_ef(bch, eb1, el1)

            wait_gather_x(bch, xb1, gx1)
            scat_x(bch, xb1, sx1)

            @pl.when(c == 0)
            def _():
                wait_load_ef(bch, eb1, el1)
                scat_aux(bch, eb1, ea1)

            @pl.when(c != 0)
            def _():
                @pl.when(p > 0)
                def _():
                    wait_scat_aux(bch, onesbuf, ea1)

                scat_aux(bch, onesbuf, ea1)

            @pl.when(p < npairs - 1)
            def _():
                wait_scat_x(a, xb0, sx0)
                gather_x(a + 2, xb0, gx0)

                @pl.when(c == 0)
                def _():
                    wait_scat_aux(a, eb0, ea0)
                    load_ef(a + 2, eb0, el0)

            return carry

        lax.fori_loop(0, npairs, body, 0)

        wait_scat_x(n_chunks - 2, xb0, sx0)
        wait_scat_x(n_chunks - 1, xb1, sx1)

        @pl.when(c == 0)
        def _():
            wait_scat_aux(n_chunks - 2, eb0, ea0)
            wait_scat_aux(n_chunks - 1, eb1, ea1)

        @pl.when(c != 0)
        def _():
            wait_scat_aux(n_chunks - 2, onesbuf, ea0)
            wait_scat_aux(n_chunks - 1, onesbuf, ea1)

        plsc.subcore_barrier()

        pltpu.sync_copy(acc_x.at[pl.ds(base, stripe)],
                        outx_hbm.at[c, pl.ds(base, stripe)])
        pltpu.sync_copy(acc_a.at[pl.ds(base, stripe)],
                        outa_hbm.at[c, pl.ds(base, stripe)])

    return sc_kernel(xcat, src3, dst3, ef2, ones_rows, zer_x, zer_e)


def _tc_body(d_half, px_ref, pa_ref, wt_ref, b_ref, out_ref):
    se = pa_ref[0]
    cnt = pa_ref[1][:, 0:1]
    acc = jnp.dot(px_ref[0], wt_ref[:d_half],
                  preferred_element_type=jnp.float32,
                  precision=lax.Precision.HIGHEST)
    acc = acc + jnp.dot(px_ref[1], wt_ref[d_half:2 * d_half],
                        preferred_element_type=jnp.float32,
                        precision=lax.Precision.HIGHEST)
    acc = acc + jnp.dot(se, wt_ref[2 * d_half:],
                        preferred_element_type=jnp.float32,
                        precision=lax.Precision.HIGHEST)
    acc = acc + cnt * b_ref[...]
    out_ref[...] = acc / jnp.maximum(cnt, 1.0)


def kernel(x, edge_index, edge_features, W, b):
    n_nodes, d_feat = x.shape
    n_edges = edge_index.shape[1]
    d_edge = edge_features.shape[1]
    out_dim = W.shape[0]
    d_half = d_feat // 2

    if n_edges % CHUNK:
        pad_e = CHUNK - n_edges % CHUNK
        edge_features = jnp.concatenate(
            [edge_features, jnp.zeros((pad_e, d_edge), edge_features.dtype)])
        edge_index = jnp.concatenate(
            [edge_index, jnp.zeros((2, pad_e), edge_index.dtype)], axis=1)
        n_edges += pad_e
    n_real_chunks = n_edges // CHUNK
    n_chunks_tot = -(-n_real_chunks // (2 * NUM_SUBCORES)) * 2 * NUM_SUBCORES
    n_chunks = n_chunks_tot // NUM_SUBCORES
    pad = n_chunks_tot * CHUNK - n_edges
    n_acc = -(-(n_nodes + 1) // 1280) * 1280
    stripe = n_acc // NUM_SUBCORES

    src = edge_index[0].astype(jnp.int32)
    dst = edge_index[1].astype(jnp.int32)
    src_p = jnp.concatenate([src, jnp.zeros((pad,), jnp.int32)])
    src3 = jnp.stack([src_p, src_p + n_nodes]).reshape(
        NUM_CORES, n_chunks_tot, CHUNK)
    dst3 = jnp.concatenate(
        [dst, jnp.full((pad,), n_nodes, jnp.int32)]).reshape(
        n_chunks_tot, CHUNK)
    xcat = jnp.concatenate([x[:, :d_half], x[:, d_half:]], axis=0)
    ones_rows = jnp.zeros((CHUNK, 16), jnp.float32).at[:, 0].set(1.0)
    zer_x = jnp.zeros((stripe, d_half), jnp.float32)
    zer_e = jnp.zeros((stripe, 16), jnp.float32)

    px, pa = _sc_segment_sums(n_acc, n_chunks, n_real_chunks, d_half, d_edge,
                              xcat.astype(jnp.float32), src3, dst3,
                              edge_features.astype(jnp.float32),
                              ones_rows, zer_x, zer_e)

    wt = W.T.astype(jnp.float32)
    b2 = b.astype(jnp.float32).reshape(1, out_dim)

    blk = 1024
    grid = n_acc // blk
    out_full = pl.pallas_call(
        functools.partial(_tc_body, d_half),
        grid=(grid,),
        in_specs=[
            pl.BlockSpec((NUM_CORES, blk, d_half), lambda i: (0, i, 0)),
            pl.BlockSpec((NUM_CORES, blk, 16), lambda i: (0, i, 0)),
            pl.BlockSpec((d_feat + d_edge, out_dim), lambda i: (0, 0)),
            pl.BlockSpec((1, out_dim), lambda i: (0, 0)),
        ],
        out_specs=pl.BlockSpec((blk, out_dim), lambda i: (i, 0)),
        out_shape=jax.ShapeDtypeStruct((n_acc, out_dim), jnp.float32),
    )(px, pa, wt, b2)

    return out_full[:n_nodes]
